# baseline (speedup 1.0000x reference)
import math

import jax
import jax.numpy as jnp
import numpy as np
from jax.experimental import pallas as pl
from jax.experimental.pallas import tpu as pltpu

H = 12
DH = 4
DV = 16
SCALE = 1.0 / math.sqrt(DH)
LOG2E = 1.4426950408889634
BQ = 1024


def _np_group96():
    g = np.zeros((96, 96), np.float32)
    for h in range(H):
        g[h * 8:(h + 1) * 8, h * 8:(h + 1) * 8] = 1.0
    return g


def _np_onec():
    o = np.zeros((96, 1), np.float32)
    for h in range(H):
        o[h * 8 + DH, 0] = LOG2E
    return o


def _np_chamber_mats():
    c1 = np.zeros((24, 96), np.float32)
    o1 = np.zeros((24, 1), np.float32)
    for h in range(H):
        for j in range(2):
            c1[h * 2 + j, h * 8 + 0] = 2 * j - 1
            o1[h * 2 + j, 0] = 1 - j
    b2 = np.zeros((48, 24), np.float32)
    c2 = np.zeros((48, 96), np.float32)
    o2 = np.zeros((48, 1), np.float32)
    for h in range(H):
        for c in range(4):
            j = (c >> 1) & 1
            b2[h * 4 + c, h * 2 + (c & 1)] = 1.0
            c2[h * 4 + c, h * 8 + 1] = 2 * j - 1
            o2[h * 4 + c, 0] = 1 - j
    b3 = np.zeros((96, 48), np.float32)
    c3 = np.zeros((96, 96), np.float32)
    o3 = np.zeros((96, 1), np.float32)
    for h in range(H):
        for c in range(8):
            j = (c >> 2) & 1
            b3[h * 8 + c, h * 4 + (c & 3)] = 1.0
            c3[h * 8 + c, h * 8 + 2] = 2 * j - 1
            o3[h * 8 + c, 0] = 1 - j
    b4 = np.zeros((192, 96), np.float32)
    c4 = np.zeros((192, 96), np.float32)
    o4 = np.zeros((192, 1), np.float32)
    for h in range(H):
        for c in range(16):
            j = (c >> 3) & 1
            b4[h * 16 + c, h * 8 + (c & 7)] = 1.0
            c4[h * 16 + c, h * 8 + 3] = 2 * j - 1
            o4[h * 16 + c, 0] = 1 - j
    return c1, o1, b2, c2, o2, b3, c3, o3, b4, c4, o4


def _np_amask():
    r = np.arange(BQ)[:, None]
    c = np.arange(BQ)[None, :]
    return np.where(r > c, np.float32(-1e30), np.float32(0.0))


_G96 = _np_group96()
_ONEC = _np_onec()
_CH = _np_chamber_mats()
_AMASK = _np_amask()
_EXPD = np.kron(np.eye(H, dtype=np.float32),
                np.ones((DV, 1), np.float32))


def _fused_kernel(x_ref, wq_ref, wk_ref, wv_ref, bdnt_ref, bdr_ref, g_ref,
                  onec_ref, c1_ref, o1_ref, b2_ref, c2_ref, o2_ref, b3_ref,
                  c3_ref, o3_ref, b4_ref, c4_ref, o4_ref, cbe_ref, amask_ref,
                  exp_ref, wout_ref, y_ref, ot_ref, l_ref):
    f32 = jnp.float32
    bf16 = jnp.bfloat16
    x = x_ref[...]
    g = g_ref[...]
    t = x.shape[0]

    def dgt(a, b):
        return jax.lax.dot_general(a, b, (((1,), (1,)), ((), ())),
                                   preferred_element_type=f32)

    def mm(a, b, prec=f32):
        return jnp.dot(a, b, preferred_element_type=prec)

    def norm_t(a):
        n2 = mm(g, a * a)
        return a / jnp.maximum(jnp.sqrt(n2), 1e-12)

    qt = norm_t(dgt(wq_ref[...], x))
    kt = norm_t(dgt(wk_ref[...], x))
    vt = dgt(wv_ref[...], x)
    qt = norm_t(mm(bdnt_ref[...], qt)) * (SCALE * LOG2E) + onec_ref[...]

    ssg = jax.nn.sigmoid(mm(bdr_ref[...], kt) * 3.0)
    u = mm(c1_ref[...], ssg) + o1_ref[...]
    u = mm(b2_ref[...], u) * (mm(c2_ref[...], ssg) + o2_ref[...])
    u = mm(b3_ref[...], u) * (mm(c3_ref[...], ssg) + o3_ref[...])
    u = mm(b4_ref[...], u) * (mm(c4_ref[...], ssg) + o4_ref[...])
    kt = kt + mm(cbe_ref[...], u)

    qtb = qt.astype(bf16)
    ktb = kt.astype(bf16)
    vtb = vt.astype(bf16)
    amask = amask_ref[...]
    ones_row = jnp.ones((1, t), bf16)

    def sdot(ks, qs):
        s = jax.lax.dot_general(ks, qs, (((0,), (0,)), ((), ())),
                                preferred_element_type=f32)
        return s.astype(bf16)

    for h in range(H):
        qh = qtb[h * 8:(h + 1) * 8, :]
        kh = ktb[h * 8:(h + 1) * 8, :]
        vh = jnp.concatenate([vtb[h * DV:(h + 1) * DV, :], ones_row], axis=0)
        for qb in range(t // BQ):
            lo = qb * BQ
            qblk = qh[:, lo:lo + BQ]
            p1 = jnp.exp2(sdot(kh[:, lo:lo + BQ], qblk) + amask)
            oa = mm(vh[:, lo:lo + BQ], p1)
            if qb:
                p0 = jnp.exp2(sdot(kh[:, :lo], qblk))
                oa = oa + mm(vh[:, :lo], p0)
            ot_ref[h * DV:(h + 1) * DV, lo:lo + BQ] = oa[:DV, :]
            l_ref[h:h + 1, lo:lo + BQ] = oa[DV:DV + 1, :]
    lbig = mm(exp_ref[...], 1.0 / l_ref[...])
    y_ref[...] = jax.lax.dot_general(ot_ref[...] * lbig, wout_ref[...],
                                     (((0,), (0,)), ((), ())),
                                     preferred_element_type=f32)


def kernel(x, Wq, Wk, Wv, Wout, W_nudge, chamber_bonus, simple_roots):
    b, t, d_model = x.shape
    x2 = x.reshape(t, d_model)

    r4 = jnp.asarray(_R4)
    p4 = jnp.asarray(_P4)
    q16 = jnp.asarray(_Q16)
    bdnt = jnp.einsum('hde,hep,hdc->pc', W_nudge, r4, r4)
    bdr = jnp.einsum('rd,hrp,hdc->pc', simple_roots, r4, r4)
    cbe = jnp.einsum('hc,hp,hcq->pq', chamber_bonus, p4, q16)

    c1, o1, b2, c2, o2, b3, c3, o3, b4, c4, o4 = (jnp.asarray(m) for m in _CH)

    y = pl.pallas_call(
        _fused_kernel,
        out_shape=jax.ShapeDtypeStruct((t, d_model), jnp.float32),
        scratch_shapes=[pltpu.VMEM((H * DV, t), jnp.float32),
                        pltpu.VMEM((H, t), jnp.float32)],
    )(x2, _pad_rows_j(Wq), _pad_rows_j(Wk), Wv, bdnt, bdr, jnp.asarray(_G96),
      jnp.asarray(_ONEC), c1, o1, b2, c2, o2, b3, c3, o3, b4, c4, o4, cbe,
      jnp.asarray(_AMASK, dtype=jnp.bfloat16), jnp.asarray(_EXPD), Wout.T)
    return y.reshape(b, t, d_model)


_PAD = np.kron(np.eye(H, dtype=np.float32),
               np.eye(8, DH, dtype=np.float32))

_R4 = np.zeros((H, DH, 96), np.float32)
_P4 = np.zeros((H, 96), np.float32)
_Q16 = np.zeros((H, 16, 192), np.float32)
for _h in range(H):
    for _i in range(DH):
        _R4[_h, _i, _h * 8 + _i] = 1.0
    _P4[_h, _h * 8 + DH] = 1.0
    for _c in range(16):
        _Q16[_h, _c, _h * 16 + _c] = 1.0


def _pad_rows_j(w):
    return jnp.asarray(_PAD) @ w


# speedup vs baseline: 1.3242x; 1.3242x over previous
import math

import jax
import jax.numpy as jnp
import numpy as np
from jax.experimental import pallas as pl
from jax.experimental.pallas import tpu as pltpu

H = 12
DH = 4
DV = 16
SCALE = 1.0 / math.sqrt(DH)
LOG2E = 1.4426950408889634
BQ = 512


def _np_chamber_mats():
    c1 = np.zeros((24, 96), np.float32)
    o1 = np.zeros((24, 1), np.float32)
    for h in range(H):
        for j in range(2):
            c1[h * 2 + j, h * 8 + 0] = 2 * j - 1
            o1[h * 2 + j, 0] = 1 - j
    b2 = np.zeros((48, 24), np.float32)
    c2 = np.zeros((48, 96), np.float32)
    o2 = np.zeros((48, 1), np.float32)
    for h in range(H):
        for c in range(4):
            j = (c >> 1) & 1
            b2[h * 4 + c, h * 2 + (c & 1)] = 1.0
            c2[h * 4 + c, h * 8 + 1] = 2 * j - 1
            o2[h * 4 + c, 0] = 1 - j
    b3 = np.zeros((96, 48), np.float32)
    c3 = np.zeros((96, 96), np.float32)
    o3 = np.zeros((96, 1), np.float32)
    for h in range(H):
        for c in range(8):
            j = (c >> 2) & 1
            b3[h * 8 + c, h * 4 + (c & 3)] = 1.0
            c3[h * 8 + c, h * 8 + 2] = 2 * j - 1
            o3[h * 8 + c, 0] = 1 - j
    b4 = np.zeros((192, 96), np.float32)
    c4 = np.zeros((192, 96), np.float32)
    o4 = np.zeros((192, 1), np.float32)
    for h in range(H):
        for c in range(16):
            j = (c >> 3) & 1
            b4[h * 16 + c, h * 8 + (c & 7)] = 1.0
            c4[h * 16 + c, h * 8 + 3] = 2 * j - 1
            o4[h * 16 + c, 0] = 1 - j
    return c1, o1, b2, c2, o2, b3, c3, o3, b4, c4, o4


def _np_amask():
    r = np.arange(BQ)[:, None]
    c = np.arange(BQ)[None, :]
    return np.where(r > c, np.float32(-1e30), np.float32(0.0))


def _np_onec():
    o = np.zeros((96, 1), np.float32)
    for h in range(H):
        o[h * 8 + DH, 0] = LOG2E
    return o


_G96 = np.kron(np.eye(H, dtype=np.float32), np.ones((8, 8), np.float32))
_PAD = np.kron(np.eye(H, dtype=np.float32),
               np.eye(8, DH, dtype=np.float32))
_TILE4 = np.kron(np.ones((1, H), np.float32),
                 np.eye(DH, 8, dtype=np.float32))
_R8 = np.kron(np.ones((H, 1), np.float32),
              np.eye(8, DH, dtype=np.float32))
_SEL4 = np.kron(np.eye(H, dtype=np.float32),
                (np.arange(8) == DH).astype(np.float32)[:, None])
_TILE16 = np.kron(np.ones((1, H), np.float32),
                  np.eye(16, dtype=np.float32))
_BLK16 = np.kron(np.eye(H, dtype=np.float32),
                 np.ones((8, 16), np.float32))
_EXPD = np.kron(np.eye(H, dtype=np.float32),
                np.ones((DV, 1), np.float32))
_ONEC = _np_onec()
_CH = _np_chamber_mats()
_AMASK = _np_amask()


def _fused_kernel(x_ref, wq_ref, wk_ref, wv_ref, wn_ref, roots_ref, cb_ref,
                  pad_ref, tile4_ref, r8_ref, sel4_ref, tile16_ref,
                  blk16_ref, g_ref, onec_ref, c1_ref, o1_ref, b2_ref, c2_ref,
                  o2_ref, b3_ref, c3_ref, o3_ref, b4_ref, c4_ref, o4_ref,
                  amask_ref, exp_ref, wout_ref, y_ref, ot_ref, l_ref):
    f32 = jnp.float32
    bf16 = jnp.bfloat16
    x = x_ref[...]
    g = g_ref[...]
    pad = pad_ref[...]
    tile4 = tile4_ref[...]
    t = x.shape[0]

    def dgt(a, b):
        return jax.lax.dot_general(a, b, (((1,), (1,)), ((), ())),
                                   preferred_element_type=f32)

    def dtg(a, b):
        return jax.lax.dot_general(a, b, (((0,), (0,)), ((), ())),
                                   preferred_element_type=f32)

    def mm(a, b):
        return jnp.dot(a, b, preferred_element_type=f32)

    def norm_t(a):
        n2 = mm(g, a * a)
        return a / jnp.maximum(jnp.sqrt(n2), 1e-12)

    qt = norm_t(dgt(mm(pad, wq_ref[...]), x))
    kt = norm_t(dgt(mm(pad, wk_ref[...]), x))
    vt = dgt(wv_ref[...], x)

    bdt = mm(mm(pad, wn_ref[...]), tile4) * g
    qt = norm_t(dtg(bdt, qt)) * (SCALE * LOG2E) + onec_ref[...]

    bdr = mm(mm(r8_ref[...], roots_ref[...]), tile4) * g
    ssg = jax.nn.sigmoid(mm(bdr, kt) * 3.0)
    u = mm(c1_ref[...], ssg) + o1_ref[...]
    u = mm(b2_ref[...], u) * (mm(c2_ref[...], ssg) + o2_ref[...])
    u = mm(b3_ref[...], u) * (mm(c3_ref[...], ssg) + o3_ref[...])
    u = mm(b4_ref[...], u) * (mm(c4_ref[...], ssg) + o4_ref[...])
    cbe = mm(sel4_ref[...], cb_ref[...])
    cbe = mm(cbe, tile16_ref[...]) * blk16_ref[...]
    kt = kt + mm(cbe, u)

    qtb = qt.astype(bf16)
    ktb = kt.astype(bf16)
    vtb = vt.astype(bf16)
    amask = amask_ref[...]
    ones_row = jnp.ones((1, t), bf16)

    def sdot(ks, qs):
        s = jax.lax.dot_general(ks, qs, (((0,), (0,)), ((), ())),
                                preferred_element_type=f32)
        return s.astype(bf16)

    for h in range(H):
        qh = qtb[h * 8:(h + 1) * 8, :]
        kh = ktb[h * 8:(h + 1) * 8, :]
        vh = jnp.concatenate([vtb[h * DV:(h + 1) * DV, :], ones_row], axis=0)
        for qb in range(t // BQ):
            lo = qb * BQ
            qblk = qh[:, lo:lo + BQ]
            p1 = jnp.exp2(sdot(kh[:, lo:lo + BQ], qblk) + amask)
            oa = mm(vh[:, lo:lo + BQ], p1)
            if qb:
                p0 = jnp.exp2(sdot(kh[:, :lo], qblk))
                oa = oa + mm(vh[:, :lo], p0)
            ot_ref[h * DV:(h + 1) * DV, lo:lo + BQ] = oa[:DV, :]
            l_ref[h:h + 1, lo:lo + BQ] = oa[DV:DV + 1, :]
    lbig = mm(exp_ref[...], 1.0 / l_ref[...])
    y_ref[...] = jax.lax.dot_general(ot_ref[...] * lbig, wout_ref[...],
                                     (((0,), (1,)), ((), ())),
                                     preferred_element_type=f32)


def kernel(x, Wq, Wk, Wv, Wout, W_nudge, chamber_bonus, simple_roots):
    b, t, d_model = x.shape
    c1, o1, b2, c2, o2, b3, c3, o3, b4, c4, o4 = (jnp.asarray(m)
                                                  for m in _CH)
    y = pl.pallas_call(
        _fused_kernel,
        out_shape=jax.ShapeDtypeStruct((t, d_model), jnp.float32),
        scratch_shapes=[pltpu.VMEM((H * DV, t), jnp.float32),
                        pltpu.VMEM((H, t), jnp.float32)],
    )(x.reshape(t, d_model), Wq, Wk, Wv, W_nudge.reshape(H * DH, DH),
      simple_roots, chamber_bonus, jnp.asarray(_PAD), jnp.asarray(_TILE4),
      jnp.asarray(_R8), jnp.asarray(_SEL4), jnp.asarray(_TILE16),
      jnp.asarray(_BLK16), jnp.asarray(_G96), jnp.asarray(_ONEC), c1, o1,
      b2, c2, o2, b3, c3, o3, b4, c4, o4,
      jnp.asarray(_AMASK, dtype=jnp.bfloat16), jnp.asarray(_EXPD), Wout)
    return y.reshape(b, t, d_model)
